# Initial kernel scaffold; baseline (speedup 1.0000x reference)
import jax
import jax.numpy as jnp
from jax.experimental import pallas as pl


def kernel(input_ids, gene_ids, gene_table, word_table, ln_weight, ln_bias):
    raise NotImplementedError("write your pallas kernel here")


# speedup vs baseline: 3.1451x; 3.1451x over previous
import functools

import jax
import jax.numpy as jnp
from jax import lax
from jax.experimental import pallas as pl
from jax.experimental.pallas import tpu as pltpu
from jax.experimental.pallas import tpu_sc as plsc

LN_EPS = 1e-12

_NC = 2
_NS = 16
_NW = _NC * _NS

_GATHER_CHUNK = 1024


def _sc_gather_body(table_hbm, idx_hbm, out_hbm, idx_v, rows_v, sem):
    n_rows = idx_hbm.shape[0]
    per_w = n_rows // _NW
    wid = lax.axis_index("s") * _NC + lax.axis_index("c")
    base = wid * per_w

    @pl.loop(0, per_w // _GATHER_CHUNK)
    def _(i):
        off = base + i * _GATHER_CHUNK
        pltpu.sync_copy(idx_hbm.at[pl.ds(off, _GATHER_CHUNK)], idx_v)
        pltpu.async_copy(table_hbm.at[idx_v], rows_v, sem).wait()
        pltpu.sync_copy(rows_v, out_hbm.at[pl.ds(off, _GATHER_CHUNK)])


def _sc_gather(table, flat_ids):
    n_rows = flat_ids.shape[0]
    d = table.shape[1]
    mesh = plsc.VectorSubcoreMesh(core_axis_name="c", subcore_axis_name="s")
    k = pl.kernel(
        _sc_gather_body,
        out_type=jax.ShapeDtypeStruct((n_rows, d), table.dtype),
        mesh=mesh,
        scratch_types=[
            pltpu.VMEM((_GATHER_CHUNK,), jnp.int32),
            pltpu.VMEM((_GATHER_CHUNK, d), table.dtype),
            pltpu.SemaphoreType.DMA,
        ],
        compiler_params=pltpu.CompilerParams(use_tc_tiling_on_sc=False),
    )
    return k(table, flat_ids)


def _tc_body(seq_len, block_rows, x_ref, g_ref, wt_ref, w_ref, b_ref, o_ref):
    x = x_ref[...]
    g = g_ref[...]
    wt = wt_ref[...]

    add = jnp.where(g == 0, wt[0:1, :], wt[1:2, :])
    add = jnp.where(g == 2, wt[2:3, :], add)
    add = jnp.where(g == 3, wt[3:4, :], add)
    i = pl.program_id(0)
    row = i * block_rows + lax.broadcasted_iota(jnp.int32, (block_rows, 1), 0)
    pos = lax.rem(row, seq_len)
    x = x + jnp.where(pos == 0, 0.0, add)

    mu = jnp.mean(x, axis=1, keepdims=True)
    xc = x - mu
    var = jnp.mean(xc * xc, axis=1, keepdims=True)
    inv = lax.rsqrt(var + LN_EPS)
    o_ref[...] = xc * inv * w_ref[...] + b_ref[...]


def _tc_add_ln(gathered, word_ids, word_table, ln_weight, ln_bias, seq_len):
    n_rows, d = gathered.shape
    block_rows = 2048
    grid = (n_rows // block_rows,)
    body = functools.partial(_tc_body, seq_len, block_rows)
    return pl.pallas_call(
        body,
        grid=grid,
        in_specs=[
            pl.BlockSpec((block_rows, d), lambda i: (i, 0)),
            pl.BlockSpec((block_rows, 1), lambda i: (i, 0)),
            pl.BlockSpec((4, d), lambda i: (0, 0)),
            pl.BlockSpec((1, d), lambda i: (0, 0)),
            pl.BlockSpec((1, d), lambda i: (0, 0)),
        ],
        out_specs=pl.BlockSpec((block_rows, d), lambda i: (i, 0)),
        out_shape=jax.ShapeDtypeStruct((n_rows, d), jnp.float32),
    )(gathered, word_ids, word_table, ln_weight, ln_bias)


def kernel(input_ids, gene_ids, gene_table, word_table, ln_weight, ln_bias):
    b, s = input_ids.shape
    d = gene_table.shape[1]
    n_rows = b * s

    flat_ids = input_ids.reshape(n_rows).astype(jnp.int32)
    gathered = _sc_gather(gene_table, flat_ids)

    g_pad = jnp.concatenate(
        [jnp.zeros((b, 1), gene_ids.dtype), gene_ids], axis=1
    ).astype(jnp.int32).reshape(n_rows, 1)

    out = _tc_add_ln(
        gathered,
        g_pad,
        word_table,
        ln_weight.reshape(1, d),
        ln_bias.reshape(1, d),
        s,
    )
    return out.reshape(b, s, d)


# speedup vs baseline: 3.7836x; 1.2030x over previous
import functools

import jax
import jax.numpy as jnp
from jax import lax
from jax.experimental import pallas as pl
from jax.experimental.pallas import tpu as pltpu
from jax.experimental.pallas import tpu_sc as plsc

LN_EPS = 1e-12

_NC = 2
_NS = 16
_NW = _NC * _NS

_GATHER_CHUNK = 512


def _sc_gather_body(table_hbm, idx_hbm, out_hbm,
                    idx_v0, idx_v1, rows_v0, rows_v1,
                    gsem0, gsem1, osem0, osem1):
    n_rows = idx_hbm.shape[0]
    per_w = n_rows // _NW
    wid = lax.axis_index("s") * _NC + lax.axis_index("c")
    base = wid * per_w
    c = _GATHER_CHUNK

    @pl.loop(0, per_w // c, step=2)
    def _(i):
        off0 = base + i * c
        off1 = off0 + c
        pltpu.sync_copy(idx_hbm.at[pl.ds(off0, c)], idx_v0)
        g0 = pltpu.async_copy(table_hbm.at[idx_v0], rows_v0, gsem0)
        pltpu.sync_copy(idx_hbm.at[pl.ds(off1, c)], idx_v1)
        g1 = pltpu.async_copy(table_hbm.at[idx_v1], rows_v1, gsem1)
        g0.wait()
        o0 = pltpu.async_copy(rows_v0, out_hbm.at[pl.ds(off0, c)], osem0)
        g1.wait()
        o1 = pltpu.async_copy(rows_v1, out_hbm.at[pl.ds(off1, c)], osem1)
        o0.wait()
        o1.wait()


def _sc_gather(table, flat_ids):
    n_rows = flat_ids.shape[0]
    d = table.shape[1]
    mesh = plsc.VectorSubcoreMesh(core_axis_name="c", subcore_axis_name="s")
    k = pl.kernel(
        _sc_gather_body,
        out_type=jax.ShapeDtypeStruct((n_rows, d), table.dtype),
        mesh=mesh,
        scratch_types=[
            pltpu.VMEM((_GATHER_CHUNK,), jnp.int32),
            pltpu.VMEM((_GATHER_CHUNK,), jnp.int32),
            pltpu.VMEM((_GATHER_CHUNK, d), table.dtype),
            pltpu.VMEM((_GATHER_CHUNK, d), table.dtype),
            pltpu.SemaphoreType.DMA,
            pltpu.SemaphoreType.DMA,
            pltpu.SemaphoreType.DMA,
            pltpu.SemaphoreType.DMA,
        ],
        compiler_params=pltpu.CompilerParams(use_tc_tiling_on_sc=False),
    )
    return k(table, flat_ids)


def _tc_body(x_ref, g_ref, wt_ref, w_ref, b_ref, o_ref):
    x = x_ref[...]
    ids = g_ref[0]
    wt5 = wt_ref[...]

    k_iota = lax.broadcasted_iota(jnp.int32, (5, x.shape[0]), 0)
    oh_t = (ids == k_iota).astype(jnp.float32)
    add = lax.dot_general(
        oh_t, wt5,
        dimension_numbers=(((0,), (0,)), ((), ())),
        preferred_element_type=jnp.float32,
    )
    x = x + add

    mu = jnp.mean(x, axis=1, keepdims=True)
    xc = x - mu
    var = jnp.mean(xc * xc, axis=1, keepdims=True)
    inv = lax.rsqrt(var + LN_EPS)
    o_ref[...] = xc * inv * w_ref[...] + b_ref[...]


def _tc_add_ln(gathered, word_ids3, word_table5, ln_weight, ln_bias):
    n_rows, d = gathered.shape
    block_rows = 2048
    grid = (n_rows // block_rows,)
    return pl.pallas_call(
        _tc_body,
        grid=grid,
        in_specs=[
            pl.BlockSpec((block_rows, d), lambda i: (i, 0)),
            pl.BlockSpec((1, 1, block_rows), lambda i: (i, 0, 0)),
            pl.BlockSpec((5, d), lambda i: (0, 0)),
            pl.BlockSpec((1, d), lambda i: (0, 0)),
            pl.BlockSpec((1, d), lambda i: (0, 0)),
        ],
        out_specs=pl.BlockSpec((block_rows, d), lambda i: (i, 0)),
        out_shape=jax.ShapeDtypeStruct((n_rows, d), jnp.float32),
    )(gathered, word_ids3, word_table5, ln_weight, ln_bias)


def kernel(input_ids, gene_ids, gene_table, word_table, ln_weight, ln_bias):
    b, s = input_ids.shape
    d = gene_table.shape[1]
    n_rows = b * s
    block_rows = 2048

    flat_ids = input_ids.reshape(n_rows).astype(jnp.int32)
    gathered = _sc_gather(gene_table, flat_ids)

    g_pad = jnp.concatenate(
        [jnp.full((b, 1), 4, jnp.int32), gene_ids.astype(jnp.int32)], axis=1
    ).reshape(n_rows // block_rows, 1, block_rows)
    wt5 = jnp.concatenate(
        [word_table, jnp.zeros((1, d), word_table.dtype)], axis=0
    )

    out = _tc_add_ln(
        gathered,
        g_pad,
        wt5,
        ln_weight.reshape(1, d),
        ln_bias.reshape(1, d),
    )
    return out.reshape(b, s, d)


# speedup vs baseline: 4.2855x; 1.1326x over previous
import functools

import jax
import jax.numpy as jnp
from jax import lax
from jax.experimental import pallas as pl
from jax.experimental.pallas import tpu as pltpu
from jax.experimental.pallas import tpu_sc as plsc

LN_EPS = 1e-12

_NC = 2
_NS = 16
_NW = _NC * _NS

_GATHER_CHUNK = 512
_LN_ROWS = 4096


def _sc_gather_body(table_hbm, idx_hbm, out_hbm,
                    idx_v0, idx_v1, rows_v0, rows_v1,
                    gsem0, gsem1, osem0, osem1):
    n_rows = idx_hbm.shape[0]
    per_w = n_rows // _NW
    wid = lax.axis_index("s") * _NC + lax.axis_index("c")
    base = wid * per_w
    c = _GATHER_CHUNK

    @pl.loop(0, per_w // c, step=2)
    def _(i):
        off0 = base + i * c
        off1 = off0 + c
        pltpu.sync_copy(idx_hbm.at[pl.ds(off0, c)], idx_v0)
        g0 = pltpu.async_copy(table_hbm.at[idx_v0], rows_v0, gsem0)
        pltpu.sync_copy(idx_hbm.at[pl.ds(off1, c)], idx_v1)
        g1 = pltpu.async_copy(table_hbm.at[idx_v1], rows_v1, gsem1)
        g0.wait()
        o0 = pltpu.async_copy(rows_v0, out_hbm.at[pl.ds(off0, c)], osem0)
        g1.wait()
        o1 = pltpu.async_copy(rows_v1, out_hbm.at[pl.ds(off1, c)], osem1)
        o0.wait()
        o1.wait()


def _sc_gather(table, flat_ids):
    n_rows = flat_ids.shape[0]
    d = table.shape[1]
    mesh = plsc.VectorSubcoreMesh(core_axis_name="c", subcore_axis_name="s")
    k = pl.kernel(
        _sc_gather_body,
        out_type=jax.ShapeDtypeStruct((n_rows, d), table.dtype),
        mesh=mesh,
        scratch_types=[
            pltpu.VMEM((_GATHER_CHUNK,), jnp.int32),
            pltpu.VMEM((_GATHER_CHUNK,), jnp.int32),
            pltpu.VMEM((_GATHER_CHUNK, d), table.dtype),
            pltpu.VMEM((_GATHER_CHUNK, d), table.dtype),
            pltpu.SemaphoreType.DMA,
            pltpu.SemaphoreType.DMA,
            pltpu.SemaphoreType.DMA,
            pltpu.SemaphoreType.DMA,
        ],
        compiler_params=pltpu.CompilerParams(use_tc_tiling_on_sc=False),
    )
    return k(table, flat_ids)


def _tc_body(xp_ref, g_ref, wt_ref, w_ref, b_ref, o_ref):
    xp = xp_ref[...]
    pid = g_ref[0]
    w20 = wt_ref[...]

    r2 = xp.shape[0]
    k_iota = lax.broadcasted_iota(jnp.int32, (20, r2), 0)
    oh_t = (pid == k_iota).astype(jnp.float32)
    add = lax.dot_general(
        oh_t, w20,
        dimension_numbers=(((0,), (0,)), ((), ())),
        preferred_element_type=jnp.float32,
    )
    x = xp + add

    lane = lax.broadcasted_iota(jnp.int32, (r2, 128), 1)
    in_a = lane < 64
    zero = jnp.zeros_like(x)
    sum_a = jnp.sum(jnp.where(in_a, x, zero), axis=1, keepdims=True)
    sum_t = jnp.sum(x, axis=1, keepdims=True)
    mu = jnp.where(in_a, sum_a, sum_t - sum_a) * (1.0 / 64.0)
    xc = x - mu
    sq = xc * xc
    sq_a = jnp.sum(jnp.where(in_a, sq, zero), axis=1, keepdims=True)
    sq_t = jnp.sum(sq, axis=1, keepdims=True)
    var = jnp.where(in_a, sq_a, sq_t - sq_a) * (1.0 / 64.0)
    inv = lax.rsqrt(var + LN_EPS)
    o_ref[...] = xc * inv * w_ref[...] + b_ref[...]


def _tc_add_ln(packed, pair_ids3, w20, ln_w2, ln_b2):
    n2 = packed.shape[0]
    r2 = _LN_ROWS // 2
    grid = (n2 // r2,)
    return pl.pallas_call(
        _tc_body,
        grid=grid,
        in_specs=[
            pl.BlockSpec((r2, 128), lambda i: (i, 0)),
            pl.BlockSpec((1, 1, r2), lambda i: (i, 0, 0)),
            pl.BlockSpec((20, 128), lambda i: (0, 0)),
            pl.BlockSpec((1, 128), lambda i: (0, 0)),
            pl.BlockSpec((1, 128), lambda i: (0, 0)),
        ],
        out_specs=pl.BlockSpec((r2, 128), lambda i: (i, 0)),
        out_shape=jax.ShapeDtypeStruct((n2, 128), jnp.float32),
    )(packed, pair_ids3, w20, ln_w2, ln_b2)


def kernel(input_ids, gene_ids, gene_table, word_table, ln_weight, ln_bias):
    b, s = input_ids.shape
    v, d = gene_table.shape
    n_rows = b * s

    flat_ids = input_ids.reshape(n_rows).astype(jnp.int32)
    gathered = _sc_gather(gene_table, flat_ids)
    packed = gathered.reshape(n_rows // 2, 128)

    g_full = jnp.concatenate(
        [jnp.full((b, 1), 4, jnp.int32), gene_ids.astype(jnp.int32)], axis=1
    )
    ga = g_full[:, 0::2]
    gb = g_full[:, 1::2]
    r2 = _LN_ROWS // 2
    pair_ids3 = (ga * 4 + gb).reshape(n_rows // _LN_ROWS, 1, r2)

    wt5 = jnp.concatenate(
        [word_table, jnp.zeros((1, d), word_table.dtype)], axis=0
    )
    a_idx = jnp.arange(20) // 4
    b_idx = jnp.arange(20) % 4
    w20 = jnp.concatenate([wt5[a_idx], word_table[b_idx]], axis=1)

    ln_w2 = jnp.concatenate([ln_weight, ln_weight]).reshape(1, 2 * d)
    ln_b2 = jnp.concatenate([ln_bias, ln_bias]).reshape(1, 2 * d)

    out = _tc_add_ln(packed, pair_ids3, w20, ln_w2, ln_b2)
    return out.reshape(b, s, d)
